# Initial kernel scaffold; baseline (speedup 1.0000x reference)
import jax
import jax.numpy as jnp
from jax.experimental import pallas as pl


def kernel(x, edge_index_r0, edge_index_r1, edge_index_r2, W1_r0, al1_r0, ar1_r0, W2_r0, al2_r0, ar2_r0, W1_r1, al1_r1, ar1_r1, W2_r1, al2_r1, ar2_r1, W1_r2, al1_r2, ar1_r2, W2_r2, al2_r2, ar2_r2, Wq1, bq1, q1, Wq2, bq2, q2):
    raise NotImplementedError("write your pallas kernel here")


# speedup vs baseline: 1.1857x; 1.1857x over previous
import functools

import jax
import jax.numpy as jnp
from jax.experimental import pallas as pl

N = 50000
E = 200000
IN, HID, OUT = 128, 64, 16
H1, H2 = 3, 1
NEG_SLOPE = 0.2
BN = 2000


def _feat1_body(x_ref, w0, w1, w2, a0, a1, a2,
                f0, f1, f2, e0, e1, e2):
    xb = x_ref[...]
    for w, a, f, e in ((w0, a0, f0, e0), (w1, a1, f1, e1), (w2, a2, f2, e2)):
        feat = jnp.dot(xb, w[...], preferred_element_type=jnp.float32)
        f[...] = feat
        e[...] = jnp.dot(feat, a[...], preferred_element_type=jnp.float32)


def _feat_layer(x, Ws, ALRs, dout):
    din = x.shape[1]
    grid = (N // BN,)
    in_specs = [pl.BlockSpec((BN, din), lambda i: (i, 0))]
    for _ in range(3):
        in_specs.append(pl.BlockSpec((din, dout), lambda i: (0, 0)))
    for _ in range(3):
        in_specs.append(pl.BlockSpec((dout, 16), lambda i: (0, 0)))
    out_specs = ([pl.BlockSpec((BN, dout), lambda i: (i, 0))] * 3
                 + [pl.BlockSpec((BN, 16), lambda i: (i, 0))] * 3)
    out_shape = ([jax.ShapeDtypeStruct((N, dout), jnp.float32)] * 3
                 + [jax.ShapeDtypeStruct((N, 16), jnp.float32)] * 3)
    return pl.pallas_call(
        _feat1_body, grid=grid, in_specs=in_specs, out_specs=out_specs,
        out_shape=out_shape,
    )(x, Ws[0], Ws[1], Ws[2], ALRs[0], ALRs[1], ALRs[2])


def _make_alr(al, ar, dout, heads):
    dhead = dout // heads
    m = jnp.zeros((dout, 16), jnp.float32)
    for h in range(heads):
        m = m.at[h * dhead:(h + 1) * dhead, h].set(al[h])
        m = m.at[h * dhead:(h + 1) * dhead, 8 + h].set(ar[h])
    return m


def _wsum_body(h0, h1, h2, wq, bq, q, out):
    i = pl.program_id(0)

    @pl.when(i == 0)
    def _():
        out[...] = jnp.zeros_like(out)

    acc = out[...]
    rows = jax.lax.broadcasted_iota(jnp.int32, (8, 128), 0)
    cols = jax.lax.broadcasted_iota(jnp.int32, (8, 128), 1)
    for r, href in enumerate((h0, h1, h2)):
        proj = jnp.tanh(jnp.dot(href[...], wq[...],
                                preferred_element_type=jnp.float32) + bq[...])
        w = jnp.dot(proj, q[...], preferred_element_type=jnp.float32)
        mask = ((rows == r) & (cols == 0)).astype(jnp.float32)
        acc = acc + jnp.sum(w) * mask
    out[...] = acc


def _sem_wsum(hs, Wq, bq, q, d):
    grid = (N // BN,)
    in_specs = [pl.BlockSpec((BN, d), lambda i: (i, 0)) for _ in range(3)]
    in_specs.append(pl.BlockSpec((d, d), lambda i: (0, 0)))
    in_specs.append(pl.BlockSpec((1, d), lambda i: (0, 0)))
    in_specs.append(pl.BlockSpec((d, 1), lambda i: (0, 0)))
    out = pl.pallas_call(
        _wsum_body, grid=grid, in_specs=in_specs,
        out_specs=pl.BlockSpec((8, 128), lambda i: (0, 0)),
        out_shape=jax.ShapeDtypeStruct((8, 128), jnp.float32),
    )(hs[0], hs[1], hs[2], Wq, bq.reshape(1, d), q.reshape(d, 1))
    return out


def _combine_body(h0, h1, h2, ws, out, *, relu):
    w = ws[0:3, 0:1] * (1.0 / N)
    m = jnp.max(w)
    b = jnp.exp(w - m)
    beta = b / jnp.sum(b)
    acc = (beta[0, 0] * h0[...] + beta[1, 0] * h1[...]
           + beta[2, 0] * h2[...])
    if relu:
        acc = jnp.maximum(acc, 0.0)
    out[...] = acc


def _sem_combine(hs, wsum, d, relu):
    grid = (N // BN,)
    in_specs = [pl.BlockSpec((BN, d), lambda i: (i, 0)) for _ in range(3)]
    in_specs.append(pl.BlockSpec((8, 128), lambda i: (0, 0)))
    return pl.pallas_call(
        functools.partial(_combine_body, relu=relu),
        grid=grid, in_specs=in_specs,
        out_specs=pl.BlockSpec((BN, d), lambda i: (i, 0)),
        out_shape=jax.ShapeDtypeStruct((N, d), jnp.float32),
    )(hs[0], hs[1], hs[2], wsum)


def _gat_edges(feat, elr, src, dst, heads, dhead):
    el = elr[:, 0:heads]
    er = elr[:, 8:8 + heads]
    e = el[src] + er[dst]
    e = jnp.where(e >= 0, e, NEG_SLOPE * e)
    ex = jnp.exp(e)
    s = jax.ops.segment_sum(ex, dst, num_segments=N)
    fsrc = feat[src].reshape(E, heads, dhead)
    seg = jax.ops.segment_sum(fsrc * ex[:, :, None], dst, num_segments=N)
    out = seg / (s[:, :, None] + 1e-9)
    return out.mean(axis=1)


def kernel(x, edge_index_r0, edge_index_r1, edge_index_r2,
           W1_r0, al1_r0, ar1_r0, W2_r0, al2_r0, ar2_r0,
           W1_r1, al1_r1, ar1_r1, W2_r1, al2_r1, ar2_r1,
           W1_r2, al1_r2, ar1_r2, W2_r2, al2_r2, ar2_r2,
           Wq1, bq1, q1, Wq2, bq2, q2):
    edges = [edge_index_r0, edge_index_r1, edge_index_r2]
    W1 = [W1_r0, W1_r1, W1_r2]
    ALR1 = [_make_alr(al, ar, H1 * HID, H1)
            for al, ar in ((al1_r0, ar1_r0), (al1_r1, ar1_r1), (al1_r2, ar1_r2))]
    W2 = [W2_r0, W2_r1, W2_r2]
    ALR2 = [_make_alr(al, ar, H2 * OUT, H2)
            for al, ar in ((al2_r0, ar2_r0), (al2_r1, ar2_r1), (al2_r2, ar2_r2))]

    f0, f1, f2, e0, e1, e2 = _feat_layer(x, W1, ALR1, H1 * HID)
    feats1 = [f0, f1, f2]
    elrs1 = [e0, e1, e2]
    hs1 = [_gat_edges(feats1[r], elrs1[r], edges[r][0], edges[r][1], H1, HID)
           for r in range(3)]
    ws1 = _sem_wsum(hs1, Wq1, bq1, q1, HID)
    h = _sem_combine(hs1, ws1, HID, relu=True)

    f0, f1, f2, e0, e1, e2 = _feat_layer(h, W2, ALR2, H2 * OUT)
    feats2 = [f0, f1, f2]
    elrs2 = [e0, e1, e2]
    hs2 = [_gat_edges(feats2[r], elrs2[r], edges[r][0], edges[r][1], H2, OUT)
           for r in range(3)]
    ws2 = _sem_wsum(hs2, Wq2, bq2, q2, OUT)
    return _sem_combine(hs2, ws2, OUT, relu=False)


# speedup vs baseline: 23.3541x; 19.6969x over previous
import functools

import jax
import jax.numpy as jnp
from jax import lax
from jax.experimental import pallas as pl
from jax.experimental.pallas import tpu as pltpu
from jax.experimental.pallas import tpu_sc as plsc

N = 50000
E = 200000
IN, HID, OUT = 128, 64, 16
H1, H2 = 3, 1
NEG_SLOPE = 0.2
BN = 2000

NTILES = 16
EPT = 12512
NCH = EPT // 16
G = 128
NPAD = 65536


def _feat_body(x_ref, w0, w1, w2, a0, a1, a2, f0, f1, f2, e0, e1, e2):
    xb = x_ref[...]
    for w, a, f, e in ((w0, a0, f0, e0), (w1, a1, f1, e1), (w2, a2, f2, e2)):
        feat = jnp.dot(xb, w[...], preferred_element_type=jnp.float32)
        f[...] = feat
        e[...] = jnp.dot(feat, a[...], preferred_element_type=jnp.float32)


def _feat_layer(x, Ws, ALRs, dout):
    din = x.shape[1]
    grid = (N // BN,)
    in_specs = [pl.BlockSpec((BN, din), lambda i: (i, 0))]
    for _ in range(3):
        in_specs.append(pl.BlockSpec((din, dout), lambda i: (0, 0)))
    for _ in range(3):
        in_specs.append(pl.BlockSpec((dout, 16), lambda i: (0, 0)))
    out_specs = ([pl.BlockSpec((BN, dout), lambda i: (i, 0))] * 3
                 + [pl.BlockSpec((BN, 16), lambda i: (i, 0))] * 3)
    out_shape = ([jax.ShapeDtypeStruct((N, dout), jnp.float32)] * 3
                 + [jax.ShapeDtypeStruct((N, 16), jnp.float32)] * 3)
    return pl.pallas_call(
        _feat_body, grid=grid, in_specs=in_specs, out_specs=out_specs,
        out_shape=out_shape,
    )(x, Ws[0], Ws[1], Ws[2], ALRs[0], ALRs[1], ALRs[2])


def _make_alr(al, ar, dout, heads):
    dhead = dout // heads
    m = jnp.zeros((dout, 16), jnp.float32)
    for h in range(heads):
        m = m.at[h * dhead:(h + 1) * dhead, h].set(al[h])
        m = m.at[h * dhead:(h + 1) * dhead, 8 + h].set(ar[h])
    return m


def _sem_body(g0, g1, g2, s0, s1, s2, wq, bq, q, h0, h1, h2, out,
              *, heads, dhead):
    i = pl.program_id(0)

    @pl.when(i == 0)
    def _():
        out[...] = jnp.zeros_like(out)

    acc = out[...]
    rows = lax.broadcasted_iota(jnp.int32, (8, 128), 0)
    cols = lax.broadcasted_iota(jnp.int32, (8, 128), 1)
    for r, (gr, sr, hr) in enumerate(((g0, s0, h0), (g1, s1, h1),
                                      (g2, s2, h2))):
        seg = gr[...]
        s = sr[...]
        hv = jnp.zeros((seg.shape[0], dhead), jnp.float32)
        for h in range(heads):
            inv = 1.0 / (s[:, h:h + 1] + 1e-9)
            hv = hv + seg[:, h * dhead:(h + 1) * dhead] * inv
        hv = hv * (1.0 / heads)
        hr[...] = hv
        proj = jnp.tanh(jnp.dot(hv, wq[...],
                                preferred_element_type=jnp.float32) + bq[...])
        w = jnp.dot(proj, q[...], preferred_element_type=jnp.float32)
        mask = ((rows == r) & (cols == 0)).astype(jnp.float32)
        acc = acc + jnp.sum(w) * mask
    out[...] = acc


def _sem_stage(segs, ss, Wq, bq, q, heads, dhead):
    d = heads * dhead
    grid = (N // BN,)
    in_specs = [pl.BlockSpec((BN, d), lambda i: (i, 0)) for _ in range(3)]
    in_specs += [pl.BlockSpec((BN, 16), lambda i: (i, 0)) for _ in range(3)]
    in_specs.append(pl.BlockSpec((dhead, dhead), lambda i: (0, 0)))
    in_specs.append(pl.BlockSpec((1, dhead), lambda i: (0, 0)))
    in_specs.append(pl.BlockSpec((dhead, 1), lambda i: (0, 0)))
    out_specs = ([pl.BlockSpec((BN, dhead), lambda i: (i, 0))] * 3
                 + [pl.BlockSpec((8, 128), lambda i: (0, 0))])
    out_shape = ([jax.ShapeDtypeStruct((N, dhead), jnp.float32)] * 3
                 + [jax.ShapeDtypeStruct((8, 128), jnp.float32)])
    return pl.pallas_call(
        functools.partial(_sem_body, heads=heads, dhead=dhead),
        grid=grid, in_specs=in_specs, out_specs=out_specs,
        out_shape=out_shape,
    )(segs[0], segs[1], segs[2], ss[0], ss[1], ss[2],
      Wq, bq.reshape(1, dhead), q.reshape(dhead, 1))


def _combine_body(h0, h1, h2, ws, out, *, relu):
    w = ws[0:3, 0:1] * (1.0 / N)
    m = jnp.max(w)
    b = jnp.exp(w - m)
    beta = b / jnp.sum(b)
    acc = beta[0, 0] * h0[...] + beta[1, 0] * h1[...] + beta[2, 0] * h2[...]
    if relu:
        acc = jnp.maximum(acc, 0.0)
    out[...] = acc


def _sem_combine(hs, wsum, d, relu):
    grid = (N // BN,)
    in_specs = [pl.BlockSpec((BN, d), lambda i: (i, 0)) for _ in range(3)]
    in_specs.append(pl.BlockSpec((8, 128), lambda i: (0, 0)))
    return pl.pallas_call(
        functools.partial(_combine_body, relu=relu),
        grid=grid, in_specs=in_specs,
        out_specs=pl.BlockSpec((BN, d), lambda i: (i, 0)),
        out_shape=jax.ShapeDtypeStruct((N, d), jnp.float32),
    )(hs[0], hs[1], hs[2], wsum)


def _edge_kernel(heads, dhead, binsize, bpc):
    dout = heads * dhead
    tr = binsize // NTILES
    zch = tr // G
    mesh = plsc.VectorSubcoreMesh(core_axis_name="c", subcore_axis_name="s",
                                  num_cores=2, num_subcores=NTILES)

    @functools.partial(
        pl.kernel,
        out_type=[jax.ShapeDtypeStruct((NPAD, dout), jnp.float32),
                  jax.ShapeDtypeStruct((NPAD, 16), jnp.float32)],
        mesh=mesh,
        compiler_params=pltpu.CompilerParams(use_tc_tiling_on_sc=False,
                                             needs_layout_passes=False),
        scratch_types=[
            pltpu.VMEM_SHARED((binsize, dout), jnp.float32),
            pltpu.VMEM_SHARED((binsize, 16), jnp.float32),
            pltpu.VMEM((EPT,), jnp.int32),
            pltpu.VMEM((EPT,), jnp.int32),
            pltpu.VMEM((EPT,), jnp.int32),
            pltpu.VMEM((G,), jnp.int32),
            pltpu.VMEM((G,), jnp.int32),
            pltpu.VMEM((G,), jnp.int32),
            pltpu.VMEM((G, 16), jnp.float32),
            pltpu.VMEM((G, 16), jnp.float32),
            pltpu.VMEM((G, 16), jnp.float32),
            pltpu.VMEM((G, 16), jnp.float32),
            pltpu.VMEM((G, dout), jnp.float32),
            pltpu.SemaphoreType.DMA,
            pltpu.SemaphoreType.DMA,
            pltpu.SemaphoreType.DMA,
            pltpu.SemaphoreType.DMA,
            pltpu.SemaphoreType.DMA,
        ],
    )
    def k(srcp, dstp, elr, feat, z16, zD, seg_o, s_o,
          acc, sacc, srcv, dstv, eidc, st_src, st_dstg, st_dstl,
          elrS, elrD, exrow, zero16, featb,
          semS, semD, semF, semX, semM):
        c = lax.axis_index("c")
        t = lax.axis_index("s")
        iota = lax.iota(jnp.int32, 16)
        zi = jnp.zeros((16,), jnp.int32)

        pltpu.sync_copy(z16, zero16)
        pltpu.sync_copy(z16, exrow)
        for kk in range(G // 16):
            st_src[pl.ds(kk * 16, 16)] = zi
            st_dstg[pl.ds(kk * 16, 16)] = zi
            st_dstl[pl.ds(kk * 16, 16)] = zi

        def init_eid(j, _):
            eidc[pl.ds(j * 16, 16)] = zi
            return 0
        lax.fori_loop(0, NCH, init_eid, 0)
        pltpu.sync_copy(srcp.at[t], srcv)
        pltpu.sync_copy(dstp.at[t], dstv)

        def flush(lo, b, cnt):
            for kk in range(G // 16):
                pos = b * G + kk * 16 + iota
                m = pos < cnt
                ev = eidc[pl.ds(b * G + kk * 16, 16)]
                sv = plsc.load_gather(srcv, [ev])
                dg = plsc.load_gather(dstv, [ev])
                dg = jnp.where(m, dg, lo)
                st_src[pl.ds(kk * 16, 16)] = jnp.where(m, sv, 0)
                st_dstg[pl.ds(kk * 16, 16)] = dg
                st_dstl[pl.ds(kk * 16, 16)] = dg - lo
            cpS = pltpu.async_copy(elr.at[st_src], elrS, semS)
            cpD = pltpu.async_copy(elr.at[st_dstg], elrD, semD)
            cpF = pltpu.async_copy(feat.at[st_src], featb, semF)
            cpS.wait()
            cpD.wait()
            rem = cnt - b * G

            def exbody(g, _):
                rowv = iota + g * 16
                valid = rowv < rem
                for h in range(heads):
                    hv = jnp.full((16,), h, jnp.int32)
                    elv = plsc.load_gather(elrS, [rowv, hv])
                    erv = plsc.load_gather(elrD, [rowv, hv + 8])
                    e = elv + erv
                    e = jnp.where(e >= 0, e, NEG_SLOPE * e)
                    ex = jnp.where(valid, jnp.exp(e), 0.0)
                    plsc.store_scatter(exrow, [rowv, hv], ex)
                return 0
            lax.fori_loop(0, G // 16, exbody, 0)
            cpX = pltpu.async_copy(exrow, sacc.at[st_dstl], semX, add=True)
            cpF.wait()

            def rowbody(r, _):
                rv = jnp.full((16,), r, jnp.int32)
                for h in range(heads):
                    exh = plsc.load_gather(
                        exrow, [rv, jnp.full((16,), h, jnp.int32)])
                    for q in range(dhead // 16):
                        off = h * dhead + q * 16
                        v = featb[r, pl.ds(off, 16)]
                        featb[r, pl.ds(off, 16)] = v * exh
                return 0
            lax.fori_loop(0, G, rowbody, 0)
            cpM = pltpu.async_copy(featb, acc.at[st_dstl], semM, add=True)
            cpX.wait()
            cpM.wait()

        def bin_body(bi, _):
            lo = (c * bpc + bi) * binsize
            hi = lo + binsize
            base = t * tr
            pltpu.sync_copy(zD, featb)
            for q in range(zch):
                pltpu.sync_copy(featb, acc.at[pl.ds(base + q * G, G)])
                pltpu.sync_copy(zero16, sacc.at[pl.ds(base + q * G, G)])
            plsc.subcore_barrier()

            def chunk(j, cnt):
                dv = dstv[pl.ds(j * 16, 16)]
                m = (dv >= lo) & (dv < hi)
                mi = m.astype(jnp.int32)
                pos = cnt + plsc.cumsum(mi) - 1
                plsc.store_scatter(eidc, [pos], iota + j * 16, mask=m)
                return cnt + jnp.sum(mi)
            cnt = lax.fori_loop(0, NCH, chunk, jnp.int32(0))

            nbatch = (cnt + (G - 1)) // G

            def batch(b, _):
                flush(lo, b, cnt)
                return 0
            lax.fori_loop(0, nbatch, batch, 0)
            plsc.subcore_barrier()

            pltpu.sync_copy(acc.at[pl.ds(base, tr)],
                            seg_o.at[pl.ds(lo + base, tr)])
            pltpu.sync_copy(sacc.at[pl.ds(base, tr)],
                            s_o.at[pl.ds(lo + base, tr)])
            plsc.subcore_barrier()
            return 0

        lax.fori_loop(0, bpc, bin_body, 0)

    return k


def _run_edges(srcp, dstp, elr, feat, heads, dhead, binsize, bpc):
    k = _edge_kernel(heads, dhead, binsize, bpc)
    z16 = jnp.zeros((G, 16), jnp.float32)
    zD = jnp.zeros((G, heads * dhead), jnp.float32)
    return k(srcp, dstp, elr, feat, z16, zD)


def kernel(x, edge_index_r0, edge_index_r1, edge_index_r2,
           W1_r0, al1_r0, ar1_r0, W2_r0, al2_r0, ar2_r0,
           W1_r1, al1_r1, ar1_r1, W2_r1, al2_r1, ar2_r1,
           W1_r2, al1_r2, ar1_r2, W2_r2, al2_r2, ar2_r2,
           Wq1, bq1, q1, Wq2, bq2, q2):
    edges = [edge_index_r0, edge_index_r1, edge_index_r2]
    W1 = [W1_r0, W1_r1, W1_r2]
    ALR1 = [_make_alr(al, ar, H1 * HID, H1)
            for al, ar in ((al1_r0, ar1_r0), (al1_r1, ar1_r1),
                           (al1_r2, ar1_r2))]
    W2 = [W2_r0, W2_r1, W2_r2]
    ALR2 = [_make_alr(al, ar, H2 * OUT, H2)
            for al, ar in ((al2_r0, ar2_r0), (al2_r1, ar2_r1),
                           (al2_r2, ar2_r2))]

    srcps, dstps = [], []
    for e in edges:
        srcps.append(jnp.pad(e[0], (0, NTILES * EPT - E)).reshape(NTILES, EPT))
        dstps.append(jnp.pad(e[1], (0, NTILES * EPT - E),
                             constant_values=-1).reshape(NTILES, EPT))

    f0, f1, f2, e0, e1, e2 = _feat_layer(x, W1, ALR1, H1 * HID)
    feats1 = [f0, f1, f2]
    elrs1 = [e0, e1, e2]
    segs1, ss1 = [], []
    for r in range(3):
        seg, s = _run_edges(srcps[r], dstps[r], elrs1[r], feats1[r],
                            H1, HID, 4096, 7)
        segs1.append(seg)
        ss1.append(s)
    ha, hb, hc, ws1 = _sem_stage(segs1, ss1, Wq1, bq1, q1, H1, HID)
    h = _sem_combine([ha, hb, hc], ws1, HID, relu=True)

    f0, f1, f2, e0, e1, e2 = _feat_layer(h, W2, ALR2, H2 * OUT)
    feats2 = [f0, f1, f2]
    elrs2 = [e0, e1, e2]
    segs2, ss2 = [], []
    for r in range(3):
        seg, s = _run_edges(srcps[r], dstps[r], elrs2[r], feats2[r],
                            H2, OUT, 32768, 1)
        segs2.append(seg)
        ss2.append(s)
    ha, hb, hc, ws2 = _sem_stage(segs2, ss2, Wq2, bq2, q2, H2, OUT)
    return _sem_combine([ha, hb, hc], ws2, OUT, relu=False)
